# Initial kernel scaffold; baseline (speedup 1.0000x reference)
import jax
import jax.numpy as jnp
from jax.experimental import pallas as pl


def kernel(x, edge_index, edge_attr, W1a, b1a, W1b, b1b, W2a, b2a, W2b, b2b):
    raise NotImplementedError("write your pallas kernel here")


# speedup vs baseline: 1.0221x; 1.0221x over previous
import jax
import jax.numpy as jnp
from jax import lax
from jax.experimental import pallas as pl
from jax.experimental.pallas import tpu as pltpu
from jax.experimental.pallas import tpu_sc as plsc

N_NODES = 10000
N_EDGES = 160000
D_NODE = 256
D_EDGE = 16
INPUT_SIZE = D_NODE + D_EDGE
HIDDEN = INPUT_SIZE * 2

NC = 2
NS = 16
CWP = 96
NCHUNK = 6
LASTW = HIDDEN - (NCHUNK - 1) * CWP
KPC = NCHUNK // NC
EB = 80
EDGES_PER_SUB = N_EDGES // NS
NBLK = EDGES_PER_SUB // EB
ROWS_PER_SUB = N_NODES // NS
ZROWS = 125
OFFS = tuple(range(0, CWP, 16))


def _xp_body(x_ref, w_ref, b_ref, out_ref):
    h = jnp.dot(x_ref[...], w_ref[...], preferred_element_type=jnp.float32)
    h = h + b_ref[...]
    for c in range(NCHUNK - 1):
        out_ref[c, :, :] = h[:, c * CWP:(c + 1) * CWP]
    out_ref[NCHUNK - 1, :, 0:LASTW] = h[:, (NCHUNK - 1) * CWP:HIDDEN]
    out_ref[NCHUNK - 1, :, LASTW:CWP] = jnp.zeros(
        (h.shape[0], CWP - LASTW), jnp.float32)


def _make_xp(x, w1a_x, b1a_row):
    rb = 2000
    return pl.pallas_call(
        _xp_body,
        grid=(N_NODES // rb,),
        in_specs=[
            pl.BlockSpec((rb, D_NODE), lambda i: (i, 0)),
            pl.BlockSpec((D_NODE, HIDDEN), lambda i: (0, 0)),
            pl.BlockSpec((1, HIDDEN), lambda i: (0, 0)),
        ],
        out_specs=pl.BlockSpec((NCHUNK, rb, CWP), lambda i: (0, i, 0)),
        out_shape=jax.ShapeDtypeStruct((NCHUNK, N_NODES, CWP), jnp.float32),
    )(x, w1a_x, b1a_row)


def _ep_body(ea_ref, w_ref, out_ref):
    h = jnp.dot(ea_ref[...], w_ref[...], preferred_element_type=jnp.float32)
    for c in range(NCHUNK - 1):
        out_ref[c, :, :] = h[:, c * CWP:(c + 1) * CWP]
    out_ref[NCHUNK - 1, :, 0:LASTW] = h[:, (NCHUNK - 1) * CWP:HIDDEN]
    out_ref[NCHUNK - 1, :, LASTW:CWP] = jnp.ones(
        (h.shape[0], CWP - LASTW), jnp.float32)


def _make_ep(edge_attr, w1a_e):
    eb = 4000
    return pl.pallas_call(
        _ep_body,
        grid=(N_EDGES // eb,),
        in_specs=[
            pl.BlockSpec((eb, D_EDGE), lambda i: (i, 0)),
            pl.BlockSpec((D_EDGE, HIDDEN), lambda i: (0, 0)),
        ],
        out_specs=pl.BlockSpec((NCHUNK, eb, CWP), lambda i: (0, i, 0)),
        out_shape=jax.ShapeDtypeStruct((NCHUNK, N_EDGES, CWP), jnp.float32),
    )(edge_attr, w1a_e)


def _seg_body(xp_ref, ep_ref, row_ref, col_ref, s_ref,
              idx_row, idx_col, gbuf, ebuf, zrow, acc, sem):
    cid = lax.axis_index("c")
    sid = lax.axis_index("s")

    zvec = jnp.zeros((16,), jnp.float32)

    def zbody(i, _):
        for o in OFFS:
            zrow[i, pl.ds(o, 16)] = zvec
        return 0

    lax.fori_loop(0, ZROWS, zbody, 0)

    for k in range(KPC):
        chunk = KPC * cid + k

        for z in range(ROWS_PER_SUB // ZROWS):
            pltpu.sync_copy(
                zrow, acc.at[pl.ds(sid * ROWS_PER_SUB + z * ZROWS, ZROWS)])
        plsc.subcore_barrier()

        def blk(b, _):
            base = sid * EDGES_PER_SUB + b * EB
            pltpu.sync_copy(row_ref.at[pl.ds(base, EB)], idx_row)
            pltpu.sync_copy(col_ref.at[pl.ds(base, EB)], idx_col)
            for j in range(EB // 16):
                idx_row[pl.ds(j * 16, 16)] = (
                    idx_row[pl.ds(j * 16, 16)] + chunk * N_NODES)
            pltpu.async_copy(xp_ref.at[idx_row], gbuf, sem).wait()
            pltpu.sync_copy(
                ep_ref.at[pl.ds(chunk * N_EDGES + base, EB)], ebuf)

            def rowbody(i, _):
                for o in OFFS:
                    g = gbuf[i, pl.ds(o, 16)]
                    e = ebuf[i, pl.ds(o, 16)]
                    gbuf[i, pl.ds(o, 16)] = jnp.maximum(g + e, 0.0)
                return 0

            lax.fori_loop(0, EB, rowbody, 0)
            pltpu.sync_copy(gbuf, acc.at[idx_col], add=True)
            return 0

        lax.fori_loop(0, NBLK, blk, 0)
        plsc.subcore_barrier()

        pltpu.sync_copy(
            acc.at[pl.ds(sid * ROWS_PER_SUB, ROWS_PER_SUB)],
            s_ref.at[pl.ds(chunk * N_NODES + sid * ROWS_PER_SUB,
                           ROWS_PER_SUB)])


def _make_seg(xp_f, ep_f, row, col):
    mesh = plsc.VectorSubcoreMesh(
        core_axis_name="c", subcore_axis_name="s",
        num_cores=NC, num_subcores=NS)
    fn = pl.kernel(
        _seg_body,
        out_type=jax.ShapeDtypeStruct((NCHUNK * N_NODES, CWP), jnp.float32),
        mesh=mesh,
        scratch_types=[
            pltpu.VMEM((EB,), jnp.int32),
            pltpu.VMEM((EB,), jnp.int32),
            pltpu.VMEM((EB, CWP), jnp.float32),
            pltpu.VMEM((EB, CWP), jnp.float32),
            pltpu.VMEM((ZROWS, CWP), jnp.float32),
            pltpu.VMEM_SHARED((N_NODES, CWP), jnp.float32),
            pltpu.SemaphoreType.DMA,
        ],
        compiler_params=pltpu.CompilerParams(use_tc_tiling_on_sc=False),
    )
    return fn(xp_f, ep_f, row, col)


def _post_body(s_ref, x_ref, w1b_ref, b1b_ref,
               w2ax_ref, w2am_ref, b2a_ref, w2b_ref, b2b_ref, out_ref):
    s = jnp.concatenate(
        [s_ref[c] for c in range(NCHUNK - 1)]
        + [s_ref[NCHUNK - 1][:, 0:LASTW]], axis=1)
    c = s_ref[NCHUNK - 1][:, LASTW:LASTW + 1]
    sums = jnp.dot(s, w1b_ref[...], preferred_element_type=jnp.float32)
    sums = sums + c * b1b_ref[...]
    mean = sums / jnp.maximum(c, 1.0)
    h = jnp.dot(x_ref[...], w2ax_ref[...], preferred_element_type=jnp.float32)
    h = h + jnp.dot(mean, w2am_ref[...], preferred_element_type=jnp.float32)
    h = jnp.maximum(h + b2a_ref[...], 0.0)
    out = jnp.dot(h, w2b_ref[...], preferred_element_type=jnp.float32)
    out_ref[...] = out + b2b_ref[...]


def _make_post(s4, x, w1b, b1b_row, w2a_x, w2a_m, b2a_row, w2b, b2b_row):
    rb = 2000
    return pl.pallas_call(
        _post_body,
        grid=(N_NODES // rb,),
        in_specs=[
            pl.BlockSpec((NCHUNK, rb, CWP), lambda i: (0, i, 0)),
            pl.BlockSpec((rb, D_NODE), lambda i: (i, 0)),
            pl.BlockSpec((HIDDEN, HIDDEN), lambda i: (0, 0)),
            pl.BlockSpec((1, HIDDEN), lambda i: (0, 0)),
            pl.BlockSpec((D_NODE, INPUT_SIZE), lambda i: (0, 0)),
            pl.BlockSpec((HIDDEN, INPUT_SIZE), lambda i: (0, 0)),
            pl.BlockSpec((1, INPUT_SIZE), lambda i: (0, 0)),
            pl.BlockSpec((INPUT_SIZE, D_NODE), lambda i: (0, 0)),
            pl.BlockSpec((1, D_NODE), lambda i: (0, 0)),
        ],
        out_specs=pl.BlockSpec((rb, D_NODE), lambda i: (i, 0)),
        out_shape=jax.ShapeDtypeStruct((N_NODES, D_NODE), jnp.float32),
    )(s4, x, w1b, b1b_row, w2a_x, w2a_m, b2a_row, w2b, b2b_row)


def kernel(x, edge_index, edge_attr, W1a, b1a, W1b, b1b, W2a, b2a, W2b, b2b):
    ei = edge_index.astype(jnp.int32)
    row = ei[0]
    col = ei[1]

    xp4 = _make_xp(x, W1a[:D_NODE], b1a.reshape(1, HIDDEN))
    ep4 = _make_ep(edge_attr, W1a[D_NODE:])
    xp_f = xp4.reshape(NCHUNK * N_NODES, CWP)
    ep_f = ep4.reshape(NCHUNK * N_EDGES, CWP)

    s_f = _make_seg(xp_f, ep_f, row, col)
    s4 = s_f.reshape(NCHUNK, N_NODES, CWP)

    return _make_post(
        s4, x, W1b, b1b.reshape(1, HIDDEN),
        W2a[:D_NODE], W2a[D_NODE:], b2a.reshape(1, INPUT_SIZE),
        W2b, b2b.reshape(1, D_NODE))


# speedup vs baseline: 1.6552x; 1.6194x over previous
import jax
import jax.numpy as jnp
from jax import lax
from jax.experimental import pallas as pl
from jax.experimental.pallas import tpu as pltpu
from jax.experimental.pallas import tpu_sc as plsc

N_NODES = 10000
N_EDGES = 160000
D_NODE = 256
D_EDGE = 16
INPUT_SIZE = D_NODE + D_EDGE
HIDDEN = INPUT_SIZE * 2

NC = 2
NS = 16
CWP = 96
NCHUNK = 6
LASTW = HIDDEN - (NCHUNK - 1) * CWP
KPC = NCHUNK // NC
EB = 80
EDGES_PER_SUB = N_EDGES // NS
NBLK = EDGES_PER_SUB // EB
ROWS_PER_SUB = N_NODES // NS
ZROWS = 125
OFFS = tuple(range(0, CWP, 16))


def _xp_body(x_ref, w_ref, b_ref, out_ref):
    h = jnp.dot(x_ref[...], w_ref[...], preferred_element_type=jnp.float32)
    out_ref[...] = h + b_ref[...]


def _make_xp(x, w1a_x_pad, b1a_pad):
    rb = 2000
    return pl.pallas_call(
        _xp_body,
        grid=(N_NODES // rb,),
        in_specs=[
            pl.BlockSpec((rb, D_NODE), lambda i: (i, 0)),
            pl.BlockSpec((D_NODE, NCHUNK * CWP), lambda i: (0, 0)),
            pl.BlockSpec((1, NCHUNK * CWP), lambda i: (0, 0)),
        ],
        out_specs=pl.BlockSpec((rb, NCHUNK * CWP), lambda i: (i, 0)),
        out_shape=jax.ShapeDtypeStruct((N_NODES, NCHUNK * CWP), jnp.float32),
    )(x, w1a_x_pad, b1a_pad)


def _ep_body(ea_ref, w_ref, b_ref, out_ref):
    ea = ea_ref[...]
    for c in range(NCHUNK):
        h = jnp.dot(ea, w_ref[c], preferred_element_type=jnp.float32)
        out_ref[c, :, :] = h + b_ref[c]


def _make_ep(edge_attr, w1a_e_chunks, e_add_chunks):
    eb = 4000
    return pl.pallas_call(
        _ep_body,
        grid=(N_EDGES // eb,),
        in_specs=[
            pl.BlockSpec((eb, D_EDGE), lambda i: (i, 0)),
            pl.BlockSpec((NCHUNK, D_EDGE, CWP), lambda i: (0, 0, 0)),
            pl.BlockSpec((NCHUNK, 1, CWP), lambda i: (0, 0, 0)),
        ],
        out_specs=pl.BlockSpec((NCHUNK, eb, CWP), lambda i: (0, i, 0)),
        out_shape=jax.ShapeDtypeStruct((NCHUNK, N_EDGES, CWP), jnp.float32),
    )(edge_attr, w1a_e_chunks, e_add_chunks)


def _seg_body(xp_ref, ep_ref, row_ref, col_ref, s_ref,
              row_all, col_all, gb0, gb1, eb0, eb1, zrow, acc,
              sg0, sg1):
    cid = lax.axis_index("c")
    sid = lax.axis_index("s")

    zvec = jnp.zeros((16,), jnp.float32)

    def zbody(i, _):
        for o in OFFS:
            zrow[i, pl.ds(o, 16)] = zvec
        return 0

    lax.fori_loop(0, ZROWS, zbody, 0)

    gbufs = (gb0, gb1)
    ebufs = (eb0, eb1)
    sems = (sg0, sg1)

    def start_ge(b, p, chunk):
        pltpu.async_copy(xp_ref.at[row_all.at[b]], gbufs[p], sems[p])
        pltpu.async_copy(
            ep_ref.at[pl.ds(chunk * N_EDGES + sid * EDGES_PER_SUB + b * EB,
                            EB)],
            ebufs[p], sems[p])

    def wait2(p):
        dummy = ep_ref.at[pl.ds(0, EB)]
        pltpu.make_async_copy(dummy, gbufs[p], sems[p]).wait()
        pltpu.make_async_copy(dummy, ebufs[p], sems[p]).wait()

    def compute_scatter(b, p):
        gbuf, ebuf = gbufs[p], ebufs[p]

        def rowbody(i, _):
            for o in OFFS:
                gbuf[i, pl.ds(o, 16)] = jnp.maximum(
                    gbuf[i, pl.ds(o, 16)] + ebuf[i, pl.ds(o, 16)], 0.0)
            return 0

        lax.fori_loop(0, EB, rowbody, 0)
        pltpu.sync_copy(gbuf, acc.at[col_all.at[b]], add=True)

    for k in range(KPC):
        chunk = KPC * cid + k

        for z in range(ROWS_PER_SUB // ZROWS):
            pltpu.sync_copy(
                zrow, acc.at[pl.ds(sid * ROWS_PER_SUB + z * ZROWS, ZROWS)])

        pltpu.sync_copy(row_ref.at[sid], row_all)
        pltpu.sync_copy(col_ref.at[sid], col_all)

        def bias_body(i, _):
            for j in range(EB // 16):
                row_all[i, pl.ds(j * 16, 16)] = (
                    row_all[i, pl.ds(j * 16, 16)] * NCHUNK + chunk)
            return 0

        lax.fori_loop(0, NBLK, bias_body, 0)
        plsc.subcore_barrier()

        start_ge(0, 0, chunk)

        def blk(b, _):
            @pl.when(lax.rem(b, 2) == 0)
            def _():
                wait2(0)

                @pl.when(b < NBLK - 1)
                def _():
                    start_ge(b + 1, 1, chunk)
                compute_scatter(b, 0)

            @pl.when(lax.rem(b, 2) == 1)
            def _():
                wait2(1)

                @pl.when(b < NBLK - 1)
                def _():
                    start_ge(b + 1, 0, chunk)
                compute_scatter(b, 1)
            return 0

        lax.fori_loop(0, NBLK, blk, 0)
        plsc.subcore_barrier()

        pltpu.sync_copy(
            acc.at[pl.ds(sid * ROWS_PER_SUB, ROWS_PER_SUB)],
            s_ref.at[pl.ds(chunk * N_NODES + sid * ROWS_PER_SUB,
                           ROWS_PER_SUB)])


def _make_seg(xp_f, ep_f, row2, col2):
    mesh = plsc.VectorSubcoreMesh(
        core_axis_name="c", subcore_axis_name="s",
        num_cores=NC, num_subcores=NS)
    fn = pl.kernel(
        _seg_body,
        out_type=jax.ShapeDtypeStruct((NCHUNK * N_NODES, CWP), jnp.float32),
        mesh=mesh,
        scratch_types=[
            pltpu.VMEM((NBLK, EB), jnp.int32),
            pltpu.VMEM((NBLK, EB), jnp.int32),
            pltpu.VMEM((EB, CWP), jnp.float32),
            pltpu.VMEM((EB, CWP), jnp.float32),
            pltpu.VMEM((EB, CWP), jnp.float32),
            pltpu.VMEM((EB, CWP), jnp.float32),
            pltpu.VMEM((ZROWS, CWP), jnp.float32),
            pltpu.VMEM_SHARED((N_NODES, CWP), jnp.float32),
            pltpu.SemaphoreType.DMA,
            pltpu.SemaphoreType.DMA,
        ],
        compiler_params=pltpu.CompilerParams(use_tc_tiling_on_sc=False),
    )
    return fn(xp_f, ep_f, row2, col2)


def _post_body(s_ref, x_ref, w1b_ref, b1b_ref,
               w2ax_ref, w2am_ref, b2a_ref, w2b_ref, b2b_ref, out_ref):
    sums = jnp.dot(s_ref[0], w1b_ref[0], preferred_element_type=jnp.float32)
    for c in range(1, NCHUNK):
        sums = sums + jnp.dot(
            s_ref[c], w1b_ref[c], preferred_element_type=jnp.float32)
    c = s_ref[NCHUNK - 1][:, LASTW:LASTW + 1]
    sums = sums + c * b1b_ref[...]
    mean = sums / jnp.maximum(c, 1.0)
    h = jnp.dot(x_ref[...], w2ax_ref[...], preferred_element_type=jnp.float32)
    h = h + jnp.dot(mean, w2am_ref[...], preferred_element_type=jnp.float32)
    h = jnp.maximum(h + b2a_ref[...], 0.0)
    out = jnp.dot(h, w2b_ref[...], preferred_element_type=jnp.float32)
    out_ref[...] = out + b2b_ref[...]


def _make_post(s4, x, w1b, b1b_row, w2a_x, w2a_m, b2a_row, w2b, b2b_row):
    rb = 2000
    return pl.pallas_call(
        _post_body,
        grid=(N_NODES // rb,),
        in_specs=[
            pl.BlockSpec((NCHUNK, rb, CWP), lambda i: (0, i, 0)),
            pl.BlockSpec((rb, D_NODE), lambda i: (i, 0)),
            pl.BlockSpec((NCHUNK, CWP, HIDDEN), lambda i: (0, 0, 0)),
            pl.BlockSpec((1, HIDDEN), lambda i: (0, 0)),
            pl.BlockSpec((D_NODE, INPUT_SIZE), lambda i: (0, 0)),
            pl.BlockSpec((HIDDEN, INPUT_SIZE), lambda i: (0, 0)),
            pl.BlockSpec((1, INPUT_SIZE), lambda i: (0, 0)),
            pl.BlockSpec((INPUT_SIZE, D_NODE), lambda i: (0, 0)),
            pl.BlockSpec((1, D_NODE), lambda i: (0, 0)),
        ],
        out_specs=pl.BlockSpec((rb, D_NODE), lambda i: (i, 0)),
        out_shape=jax.ShapeDtypeStruct((N_NODES, D_NODE), jnp.float32),
    )(s4, x, w1b, b1b_row, w2a_x, w2a_m, b2a_row, w2b, b2b_row)


def _pad_cols(w):
    return jnp.pad(w, ((0, 0), (0, NCHUNK * CWP - HIDDEN)))


def kernel(x, edge_index, edge_attr, W1a, b1a, W1b, b1b, W2a, b2a, W2b, b2b):
    ei = edge_index.astype(jnp.int32)
    row2 = ei[0].reshape(NS, NBLK, EB)
    col2 = ei[1].reshape(NS, NBLK, EB)

    w1a_x_pad = _pad_cols(W1a[:D_NODE])
    b1a_pad = _pad_cols(b1a.reshape(1, HIDDEN))
    w1a_e = _pad_cols(W1a[D_NODE:])
    w1a_e_chunks = w1a_e.reshape(D_EDGE, NCHUNK, CWP).transpose(1, 0, 2)
    e_add_chunks = jnp.zeros(
        (NCHUNK, 1, CWP), jnp.float32).at[NCHUNK - 1, 0, LASTW:].set(1.0)
    w1b_rows = jnp.pad(W1b, ((0, NCHUNK * CWP - HIDDEN), (0, 0)))
    w1b_chunks = w1b_rows.reshape(NCHUNK, CWP, HIDDEN)

    xp_w = _make_xp(x, w1a_x_pad, b1a_pad)
    ep4 = _make_ep(edge_attr, w1a_e_chunks, e_add_chunks)
    xp_f = xp_w.reshape(NCHUNK * N_NODES, CWP)
    ep_f = ep4.reshape(NCHUNK * N_EDGES, CWP)

    s_f = _make_seg(xp_f, ep_f, row2, col2)
    s4 = s_f.reshape(NCHUNK, N_NODES, CWP)

    return _make_post(
        s4, x, w1b_chunks, b1b.reshape(1, HIDDEN),
        W2a[:D_NODE], W2a[D_NODE:], b2a.reshape(1, INPUT_SIZE),
        W2b, b2b.reshape(1, D_NODE))
